# Initial kernel scaffold; baseline (speedup 1.0000x reference)
import jax
import jax.numpy as jnp
from jax.experimental import pallas as pl


def kernel(X, Wv, bv, We, be, edge_index):
    raise NotImplementedError("write your pallas kernel here")


# speedup vs baseline: 3.0310x; 3.0310x over previous
import functools

import jax
import jax.numpy as jnp
from jax import lax
from jax.experimental import pallas as pl
from jax.experimental.pallas import tpu as pltpu, tpu_sc as plsc

NV = 10000
NE = 10000
NNZ = 320000
C = 128

NCORES = 2
NSUB = 16
NW = NCORES * NSUB
CHUNK = 80
CH_PER_TILE = 128
PAIRS_PER_TILE = CHUNK * CH_PER_TILE
NP_PAD = NW * PAIRS_PER_TILE
AW = 144
ROWS = 10240
RPT = ROWS // NSUB
NCOPY = RPT // CHUNK


def _mm_bias(x, w, b):
    M = x.shape[0]
    G = 8
    R = M // G

    def body(x_ref, w_ref, b_ref, o_ref):
        o_ref[...] = (
            lax.dot_general(x_ref[...], w_ref[...], (((1,), (1,)), ((), ())),
                            preferred_element_type=jnp.float32)
            + b_ref[...]
        )

    return pl.pallas_call(
        body,
        grid=(G,),
        in_specs=[
            pl.BlockSpec((R, C), lambda i: (i, 0)),
            pl.BlockSpec((C, C), lambda i: (0, 0)),
            pl.BlockSpec((1, C), lambda i: (0, 0)),
        ],
        out_specs=pl.BlockSpec((R, C), lambda i: (i, 0)),
        out_shape=jax.ShapeDtypeStruct((M, C), jnp.float32),
    )(x, w, b.reshape(1, C))


def _mid_transform(partials, w, b):
    G = 8
    R = ROWS // G

    def body(p_ref, w_ref, b_ref, o_ref):
        s = p_ref[0] + p_ref[1]
        cnt = jnp.maximum(s[:, C:C + 1], 1.0)
        y = jax.nn.relu(s[:, :C]) / cnt
        o_ref[...] = (
            lax.dot_general(y, w_ref[...], (((1,), (1,)), ((), ())),
                            preferred_element_type=jnp.float32)
            + b_ref[...]
        )

    return pl.pallas_call(
        body,
        grid=(G,),
        in_specs=[
            pl.BlockSpec((2, R, AW), lambda i: (0, i, 0)),
            pl.BlockSpec((C, C), lambda i: (0, 0)),
            pl.BlockSpec((1, C), lambda i: (0, 0)),
        ],
        out_specs=pl.BlockSpec((R, C), lambda i: (i, 0)),
        out_shape=jax.ShapeDtypeStruct((ROWS, C), jnp.float32),
    )(partials, w, b.reshape(1, C))


def _final_mean_relu(partials):
    G = 5
    R = NV // G

    def body(p_ref, o_ref):
        s = p_ref[0] + p_ref[1]
        cnt = jnp.maximum(s[:, C:C + 1], 1.0)
        o_ref[...] = jax.nn.relu(s[:, :C]) / cnt

    return pl.pallas_call(
        body,
        grid=(G,),
        in_specs=[pl.BlockSpec((2, R, AW), lambda i: (0, i, 0))],
        out_specs=pl.BlockSpec((R, C), lambda i: (i, 0)),
        out_shape=jax.ShapeDtypeStruct((NV, C), jnp.float32),
    )(partials)


def _sc_seg_sum(feat_aug, gidx2d, sidx2d, zeros_blk):
    mesh = plsc.VectorSubcoreMesh(core_axis_name="c", subcore_axis_name="s")

    @functools.partial(
        pl.kernel,
        out_type=jax.ShapeDtypeStruct((NCORES, ROWS, AW), jnp.float32),
        mesh=mesh,
        compiler_params=pltpu.CompilerParams(use_tc_tiling_on_sc=False),
        scratch_types=[
            pltpu.VMEM((CH_PER_TILE, CHUNK), jnp.int32),
            pltpu.VMEM((CH_PER_TILE, CHUNK), jnp.int32),
            pltpu.VMEM((CHUNK, AW), jnp.float32),
            pltpu.VMEM_SHARED((ROWS, AW), jnp.float32),
            pltpu.SemaphoreType.DMA,
        ],
    )
    def k(feat_hbm, gidx_hbm, sidx_hbm, zeros_hbm, out_hbm,
          gix_v, six_v, rows_v, acc_sh, sem):
        c = lax.axis_index("c")
        s = lax.axis_index("s")
        tile = s * NCORES + c
        row_base = tile * CH_PER_TILE

        pltpu.sync_copy(gidx_hbm.at[pl.ds(row_base, CH_PER_TILE)], gix_v)
        pltpu.sync_copy(sidx_hbm.at[pl.ds(row_base, CH_PER_TILE)], six_v)

        pltpu.sync_copy(zeros_hbm, rows_v)

        def zbody(kk, carry):
            pltpu.sync_copy(rows_v, acc_sh.at[pl.ds(s * RPT + kk * CHUNK, CHUNK)])
            return carry

        lax.fori_loop(0, NCOPY, zbody, 0)
        plsc.subcore_barrier()

        def body(j, carry):
            pltpu.async_copy(feat_hbm.at[gix_v.at[j]], rows_v, sem).wait()
            pltpu.sync_copy(rows_v, acc_sh.at[six_v.at[j]], add=True)
            return carry

        lax.fori_loop(0, CH_PER_TILE, body, 0)
        plsc.subcore_barrier()

        def obody(kk, carry):
            pltpu.sync_copy(acc_sh.at[pl.ds(s * RPT + kk * CHUNK, CHUNK)], rows_v)
            pltpu.sync_copy(rows_v, out_hbm.at[c, pl.ds(s * RPT + kk * CHUNK, CHUNK)])
            return carry

        lax.fori_loop(0, NCOPY, obody, 0)

    return k(feat_aug, gidx2d, sidx2d, zeros_blk)


def kernel(X, Wv, bv, We, be, edge_index):
    v_idx = edge_index[0].astype(jnp.int32)
    e_idx = edge_index[1].astype(jnp.int32)
    pad = NP_PAD - NNZ
    dummy = jnp.full((pad,), NV, dtype=jnp.int32)
    vg = jnp.concatenate([v_idx, dummy]).reshape(NW * CH_PER_TILE, CHUNK)
    eg = jnp.concatenate([e_idx, dummy]).reshape(NW * CH_PER_TILE, CHUNK)

    zeros_blk = jnp.zeros((CHUNK, AW), jnp.float32)
    ones_col = jnp.ones((ROWS, 1), jnp.float32)
    pad_cols = jnp.zeros((ROWS, AW - C - 1), jnp.float32)

    Xpad = jnp.concatenate([X, jnp.zeros((ROWS - NV, C), jnp.float32)], axis=0)
    Xp = _mm_bias(Xpad, Wv, bv)
    Xp_aug = jnp.concatenate([Xp, ones_col, pad_cols], axis=1)

    p1 = _sc_seg_sum(Xp_aug, vg, eg, zeros_blk)

    Yp = _mid_transform(p1, We, be)
    Yp_aug = jnp.concatenate([Yp, ones_col, pad_cols], axis=1)

    p2 = _sc_seg_sum(Yp_aug, eg, vg, zeros_blk)

    return _final_mean_relu(p2)
